# Initial kernel scaffold; baseline (speedup 1.0000x reference)
import jax
import jax.numpy as jnp
from jax.experimental import pallas as pl


def kernel(x, edge_index):
    raise NotImplementedError("write your pallas kernel here")


# speedup vs baseline: 16.0222x; 16.0222x over previous
import jax
import jax.numpy as jnp
from jax import lax
from jax.experimental import pallas as pl
from jax.experimental.pallas import tpu as pltpu
from jax.experimental.pallas import tpu_sc as plsc

N_NODES = 10000
D_FEAT = 128
N_EDGES = 320000
NUM_LAYERS = 3

NC = 2
NS = 16
NW = NC * NS
E_PER_W = N_EDGES // NW
K = 80
NCHUNK = E_PER_W // K
NSEC = 5
SECCH = NCHUNK // NSEC
N_PAD = 10240
ROWS_PER_TILE = N_PAD // NS
DEG_W = 128

_mesh = plsc.VectorSubcoreMesh(core_axis_name="c", subcore_axis_name="s")


def _sc_deg_body(dst_hbm, zeros_hbm, ones_hbm, degp_hbm,
                 dst_idx, ones_v, acc, sem):
    c = lax.axis_index("c")
    s = lax.axis_index("s")
    w = s * NC + c
    pltpu.sync_copy(ones_hbm, ones_v)
    pltpu.sync_copy(zeros_hbm, acc.at[pl.ds(s * ROWS_PER_TILE, ROWS_PER_TILE)])
    plsc.subcore_barrier()

    for sec in range(NSEC):
        pltpu.async_copy(dst_hbm.at[w, sec], dst_idx, sem).wait()

        def chunk(j, carry):
            pltpu.sync_copy(ones_v, acc.at[dst_idx.at[j]], add=True)
            return carry

        lax.fori_loop(0, SECCH, chunk, 0)

    plsc.subcore_barrier()
    pltpu.sync_copy(acc.at[pl.ds(s * ROWS_PER_TILE, ROWS_PER_TILE)],
                    degp_hbm.at[c, pl.ds(s * ROWS_PER_TILE, ROWS_PER_TILE)])


_sc_deg = pl.kernel(
    _sc_deg_body,
    out_type=jax.ShapeDtypeStruct((NC, N_PAD, DEG_W), jnp.float32),
    mesh=_mesh,
    scratch_types=[
        pltpu.VMEM((SECCH, K), jnp.int32),
        pltpu.VMEM((K, DEG_W), jnp.float32),
        pltpu.VMEM_SHARED((N_PAD, DEG_W), jnp.float32),
        pltpu.SemaphoreType.DMA,
    ],
)


def _sc_layer_body(g_hbm, src_hbm, dst_hbm, zeros_hbm, part_hbm,
                   src_idx, dst_idx, rows_a, rows_b, acc,
                   sem_a, sem_b, sem_i):
    c = lax.axis_index("c")
    s = lax.axis_index("s")
    w = s * NC + c
    pltpu.sync_copy(zeros_hbm, acc.at[pl.ds(s * ROWS_PER_TILE, ROWS_PER_TILE)])
    plsc.subcore_barrier()

    for sec in range(NSEC):
        pltpu.async_copy(src_hbm.at[w, sec], src_idx, sem_i)
        pltpu.async_copy(dst_hbm.at[w, sec], dst_idx, sem_i)
        pltpu.make_async_copy(src_hbm.at[w, sec], src_idx, sem_i).wait()
        pltpu.make_async_copy(dst_hbm.at[w, sec], dst_idx, sem_i).wait()

        pltpu.async_copy(g_hbm.at[src_idx.at[0]], rows_a, sem_a)

        def chunk(jj, carry):
            for b in range(2):
                rows, sem = (rows_a, sem_a) if b == 0 else (rows_b, sem_b)
                nrows, nsem = (rows_b, sem_b) if b == 0 else (rows_a, sem_a)
                j = jj * 2 + b
                pltpu.async_copy(g_hbm.at[src_idx.at[j + 1]], nrows, nsem)
                pltpu.make_async_copy(g_hbm.at[src_idx.at[j]], rows, sem).wait()
                pltpu.sync_copy(rows, acc.at[dst_idx.at[j]], add=True)
            return carry

        lax.fori_loop(0, (SECCH - 1) // 2, chunk, 0)
        jt = SECCH - 1
        pltpu.make_async_copy(g_hbm.at[src_idx.at[jt]], rows_a, sem_a).wait()
        pltpu.sync_copy(rows_a, acc.at[dst_idx.at[jt]], add=True)

    plsc.subcore_barrier()
    pltpu.sync_copy(acc.at[pl.ds(s * ROWS_PER_TILE, ROWS_PER_TILE)],
                    part_hbm.at[c, pl.ds(s * ROWS_PER_TILE, ROWS_PER_TILE)])


_sc_layer = pl.kernel(
    _sc_layer_body,
    out_type=jax.ShapeDtypeStruct((NC, N_PAD, D_FEAT), jnp.float32),
    mesh=_mesh,
    scratch_types=[
        pltpu.VMEM((SECCH, K), jnp.int32),
        pltpu.VMEM((SECCH, K), jnp.int32),
        pltpu.VMEM((K, D_FEAT), jnp.float32),
        pltpu.VMEM((K, D_FEAT), jnp.float32),
        pltpu.VMEM_SHARED((N_PAD, D_FEAT), jnp.float32),
        pltpu.SemaphoreType.DMA,
        pltpu.SemaphoreType.DMA,
        pltpu.SemaphoreType.DMA,
    ],
)


_RB = 2000


def _tc_prep_body(degp_ref, x_ref, dis_ref, g_ref, acc_ref):
    deg = degp_ref[0, :, 0:1] + degp_ref[1, :, 0:1]
    dis = jnp.where(deg > 0, lax.rsqrt(deg), 0.0)
    dis_b = jnp.broadcast_to(dis, (_RB, D_FEAT))
    dis_ref[...] = dis_b
    g_ref[...] = dis_b * x_ref[...]
    acc_ref[...] = x_ref[...] * (1.0 / (NUM_LAYERS + 1))


_tc_prep = pl.pallas_call(
    _tc_prep_body,
    grid=(N_NODES // _RB,),
    in_specs=[
        pl.BlockSpec((NC, _RB, DEG_W), lambda i: (0, i, 0)),
        pl.BlockSpec((_RB, D_FEAT), lambda i: (i, 0)),
    ],
    out_specs=[
        pl.BlockSpec((_RB, D_FEAT), lambda i: (i, 0)),
        pl.BlockSpec((_RB, D_FEAT), lambda i: (i, 0)),
        pl.BlockSpec((_RB, D_FEAT), lambda i: (i, 0)),
    ],
    out_shape=[
        jax.ShapeDtypeStruct((N_NODES, D_FEAT), jnp.float32),
        jax.ShapeDtypeStruct((N_NODES, D_FEAT), jnp.float32),
        jax.ShapeDtypeStruct((N_NODES, D_FEAT), jnp.float32),
    ],
)


def _tc_combine_body(p_ref, dis_ref, acc_ref, accout_ref, g_ref):
    h = dis_ref[...] * (p_ref[0] + p_ref[1])
    accout_ref[...] = acc_ref[...] + h * (1.0 / (NUM_LAYERS + 1))
    g_ref[...] = dis_ref[...] * h


_tc_combine = pl.pallas_call(
    _tc_combine_body,
    grid=(N_NODES // _RB,),
    in_specs=[
        pl.BlockSpec((NC, _RB, D_FEAT), lambda i: (0, i, 0)),
        pl.BlockSpec((_RB, D_FEAT), lambda i: (i, 0)),
        pl.BlockSpec((_RB, D_FEAT), lambda i: (i, 0)),
    ],
    out_specs=[
        pl.BlockSpec((_RB, D_FEAT), lambda i: (i, 0)),
        pl.BlockSpec((_RB, D_FEAT), lambda i: (i, 0)),
    ],
    out_shape=[
        jax.ShapeDtypeStruct((N_NODES, D_FEAT), jnp.float32),
        jax.ShapeDtypeStruct((N_NODES, D_FEAT), jnp.float32),
    ],
)


def kernel(x, edge_index):
    ei = edge_index.astype(jnp.int32)
    src_r = ei[0].reshape(NW, NSEC, SECCH, K)
    dst_r = ei[1].reshape(NW, NSEC, SECCH, K)
    zeros_deg = jnp.zeros((ROWS_PER_TILE, DEG_W), jnp.float32)
    ones_deg = jnp.ones((K, DEG_W), jnp.float32)
    zeros_feat = jnp.zeros((ROWS_PER_TILE, D_FEAT), jnp.float32)

    degp = _sc_deg(dst_r, zeros_deg, ones_deg)
    dis2d, g, acc0 = _tc_prep(degp, x)

    def step(carry, _):
        acc, g = carry
        part = _sc_layer(g, src_r, dst_r, zeros_feat)
        acc, g = _tc_combine(part, dis2d, acc)
        return (acc, g), None

    (acc, _), _ = lax.scan(step, (acc0, g), None, length=NUM_LAYERS)
    return acc


# speedup vs baseline: 16.1649x; 1.0089x over previous
import jax
import jax.numpy as jnp
from jax import lax
from jax.experimental import pallas as pl
from jax.experimental.pallas import tpu as pltpu
from jax.experimental.pallas import tpu_sc as plsc

N_NODES = 10000
D_FEAT = 128
N_EDGES = 320000
NUM_LAYERS = 3

NC = 2
NS = 16
NW = NC * NS
E_PER_W = N_EDGES // NW
K = 80
NCHUNK = E_PER_W // K
NSEC = 5
SECCH = NCHUNK // NSEC
N_PAD = 10240
ROWS_PER_TILE = N_PAD // NS
DEG_W = 128

_mesh = plsc.VectorSubcoreMesh(core_axis_name="c", subcore_axis_name="s")


def _sc_deg_body(dst_hbm, zeros_hbm, ones_hbm, degp_hbm,
                 dst_idx, ones_v, acc, sem):
    c = lax.axis_index("c")
    s = lax.axis_index("s")
    w = s * NC + c
    pltpu.sync_copy(ones_hbm, ones_v)
    pltpu.sync_copy(zeros_hbm, acc.at[pl.ds(s * ROWS_PER_TILE, ROWS_PER_TILE)])
    plsc.subcore_barrier()

    for sec in range(NSEC):
        pltpu.async_copy(dst_hbm.at[w, sec], dst_idx, sem).wait()

        def chunk(j, carry):
            pltpu.sync_copy(ones_v, acc.at[dst_idx.at[j]], add=True)
            return carry

        lax.fori_loop(0, SECCH, chunk, 0)

    plsc.subcore_barrier()
    pltpu.sync_copy(acc.at[pl.ds(s * ROWS_PER_TILE, ROWS_PER_TILE)],
                    degp_hbm.at[c, pl.ds(s * ROWS_PER_TILE, ROWS_PER_TILE)])


_sc_deg = pl.kernel(
    _sc_deg_body,
    out_type=jax.ShapeDtypeStruct((NC, N_PAD, DEG_W), jnp.float32),
    mesh=_mesh,
    scratch_types=[
        pltpu.VMEM((SECCH, K), jnp.int32),
        pltpu.VMEM((K, DEG_W), jnp.float32),
        pltpu.VMEM_SHARED((N_PAD, DEG_W), jnp.float32),
        pltpu.SemaphoreType.DMA,
    ],
)


def _sc_layer_body(g_hbm, src_hbm, dst_hbm, zeros_hbm, part_hbm,
                   src_idx, dst_idx, rows_a, rows_b, acc,
                   sem_a, sem_b, sem_i):
    c = lax.axis_index("c")
    s = lax.axis_index("s")
    w = s * NC + c
    pltpu.sync_copy(zeros_hbm, acc.at[pl.ds(s * ROWS_PER_TILE, ROWS_PER_TILE)])
    plsc.subcore_barrier()

    for sec in range(NSEC):
        pltpu.async_copy(src_hbm.at[w, sec], src_idx, sem_i)
        pltpu.async_copy(dst_hbm.at[w, sec], dst_idx, sem_i)
        pltpu.make_async_copy(src_hbm.at[w, sec], src_idx, sem_i).wait()
        pltpu.make_async_copy(dst_hbm.at[w, sec], dst_idx, sem_i).wait()

        pltpu.async_copy(g_hbm.at[src_idx.at[0]], rows_a, sem_a)

        def chunk(jj, carry):
            for b in range(2):
                rows, sem = (rows_a, sem_a) if b == 0 else (rows_b, sem_b)
                nrows, nsem = (rows_b, sem_b) if b == 0 else (rows_a, sem_a)
                j = jj * 2 + b
                pltpu.async_copy(g_hbm.at[src_idx.at[j + 1]], nrows, nsem)
                pltpu.make_async_copy(g_hbm.at[src_idx.at[j]], rows, sem).wait()
                pltpu.sync_copy(rows, acc.at[dst_idx.at[j]], add=True)
            return carry

        lax.fori_loop(0, (SECCH - 1) // 2, chunk, 0)
        jt = SECCH - 1
        pltpu.make_async_copy(g_hbm.at[src_idx.at[jt]], rows_a, sem_a).wait()
        pltpu.sync_copy(rows_a, acc.at[dst_idx.at[jt]], add=True)

    plsc.subcore_barrier()
    pltpu.sync_copy(acc.at[pl.ds(s * ROWS_PER_TILE, ROWS_PER_TILE)],
                    part_hbm.at[c, pl.ds(s * ROWS_PER_TILE, ROWS_PER_TILE)])


_sc_layer = pl.kernel(
    _sc_layer_body,
    out_type=jax.ShapeDtypeStruct((NC, N_PAD, D_FEAT), jnp.float32),
    mesh=_mesh,
    scratch_types=[
        pltpu.VMEM((SECCH, K), jnp.int32),
        pltpu.VMEM((SECCH, K), jnp.int32),
        pltpu.VMEM((K, D_FEAT), jnp.float32),
        pltpu.VMEM((K, D_FEAT), jnp.float32),
        pltpu.VMEM_SHARED((N_PAD, D_FEAT), jnp.float32),
        pltpu.SemaphoreType.DMA,
        pltpu.SemaphoreType.DMA,
        pltpu.SemaphoreType.DMA,
    ],
)


_RB = 5000


def _tc_prep_body(degp_ref, x_ref, dis_ref, g_ref, acc_ref):
    deg = degp_ref[0, :, 0:1] + degp_ref[1, :, 0:1]
    dis = jnp.where(deg > 0, lax.rsqrt(deg), 0.0)
    dis_b = jnp.broadcast_to(dis, (_RB, D_FEAT))
    dis_ref[...] = dis_b
    g_ref[...] = dis_b * x_ref[...]
    acc_ref[...] = x_ref[...] * (1.0 / (NUM_LAYERS + 1))


_tc_prep = pl.pallas_call(
    _tc_prep_body,
    grid=(N_NODES // _RB,),
    in_specs=[
        pl.BlockSpec((NC, _RB, DEG_W), lambda i: (0, i, 0)),
        pl.BlockSpec((_RB, D_FEAT), lambda i: (i, 0)),
    ],
    out_specs=[
        pl.BlockSpec((_RB, D_FEAT), lambda i: (i, 0)),
        pl.BlockSpec((_RB, D_FEAT), lambda i: (i, 0)),
        pl.BlockSpec((_RB, D_FEAT), lambda i: (i, 0)),
    ],
    out_shape=[
        jax.ShapeDtypeStruct((N_NODES, D_FEAT), jnp.float32),
        jax.ShapeDtypeStruct((N_NODES, D_FEAT), jnp.float32),
        jax.ShapeDtypeStruct((N_NODES, D_FEAT), jnp.float32),
    ],
)


def _tc_combine_body(p_ref, dis_ref, acc_ref, accout_ref, g_ref):
    h = dis_ref[...] * (p_ref[0] + p_ref[1])
    accout_ref[...] = acc_ref[...] + h * (1.0 / (NUM_LAYERS + 1))
    g_ref[...] = dis_ref[...] * h


_tc_combine = pl.pallas_call(
    _tc_combine_body,
    grid=(N_NODES // _RB,),
    in_specs=[
        pl.BlockSpec((NC, _RB, D_FEAT), lambda i: (0, i, 0)),
        pl.BlockSpec((_RB, D_FEAT), lambda i: (i, 0)),
        pl.BlockSpec((_RB, D_FEAT), lambda i: (i, 0)),
    ],
    out_specs=[
        pl.BlockSpec((_RB, D_FEAT), lambda i: (i, 0)),
        pl.BlockSpec((_RB, D_FEAT), lambda i: (i, 0)),
    ],
    out_shape=[
        jax.ShapeDtypeStruct((N_NODES, D_FEAT), jnp.float32),
        jax.ShapeDtypeStruct((N_NODES, D_FEAT), jnp.float32),
    ],
)


def kernel(x, edge_index):
    ei = edge_index.astype(jnp.int32)
    src_r = ei[0].reshape(NW, NSEC, SECCH, K)
    dst_r = ei[1].reshape(NW, NSEC, SECCH, K)
    zeros_deg = jnp.zeros((ROWS_PER_TILE, DEG_W), jnp.float32)
    ones_deg = jnp.ones((K, DEG_W), jnp.float32)
    zeros_feat = jnp.zeros((ROWS_PER_TILE, D_FEAT), jnp.float32)

    degp = _sc_deg(dst_r, zeros_deg, ones_deg)
    dis2d, g, acc0 = _tc_prep(degp, x)

    def step(carry, _):
        acc, g = carry
        part = _sc_layer(g, src_r, dst_r, zeros_feat)
        acc, g = _tc_combine(part, dis2d, acc)
        return (acc, g), None

    (acc, _), _ = lax.scan(step, (acc0, g), None, length=NUM_LAYERS)
    return acc
